# baseline (device time: 8367 ns/iter reference)
import jax
import jax.numpy as jnp
from jax import lax
from jax.experimental import pallas as pl
from jax.experimental.pallas import tpu as pltpu

N_DEV = 4
K = 8


def kernel(x):
    m_per, n = x.shape
    chunk = m_per // K

    def body(x_ref, out_ref, comm_ref, send_sems, recv_sems):
        i = pl.program_id(0)
        my_pos = lax.axis_index("i")
        barrier_sem = pltpu.get_barrier_semaphore()

        @pl.when(i == 0)
        def _():
            for d in range(1, N_DEV):
                peer = lax.rem(my_pos + d, N_DEV)
                pl.semaphore_signal(
                    barrier_sem, inc=1,
                    device_id=(peer,), device_id_type=pl.DeviceIdType.MESH,
                )
            comm_ref[0, :, :] = jnp.zeros((8, n), comm_ref.dtype)

        comm_ref[0, :, :] += jnp.sum(
            x_ref[:, :].reshape(chunk // 8, 8, n), axis=0
        )

        @pl.when(i == K - 1)
        def _():
            pl.semaphore_wait(barrier_sem, N_DEV - 1)

            sends = []
            for d in range(1, N_DEV):
                peer = lax.rem(my_pos + d, N_DEV)
                e = N_DEV - d
                rdma = pltpu.make_async_remote_copy(
                    src_ref=comm_ref.at[0],
                    dst_ref=comm_ref.at[e],
                    send_sem=send_sems.at[d - 1],
                    recv_sem=recv_sems.at[e],
                    device_id=(peer,),
                    device_id_type=pl.DeviceIdType.MESH,
                )
                rdma.start()
                sends.append(rdma)

            for e in range(1, N_DEV):
                recv = pltpu.make_async_remote_copy(
                    src_ref=comm_ref.at[0],
                    dst_ref=comm_ref.at[e],
                    send_sem=send_sems.at[0],
                    recv_sem=recv_sems.at[e],
                    device_id=(my_pos,),
                    device_id_type=pl.DeviceIdType.MESH,
                )
                recv.wait_recv()

            total = (
                comm_ref[0, :, :] + comm_ref[1, :, :]
                + comm_ref[2, :, :] + comm_ref[3, :, :]
            )
            out_ref[:, :] = jnp.sum(total, axis=0, keepdims=True)

            for rdma in sends:
                rdma.wait_send()

    return pl.pallas_call(
        body,
        grid=(K,),
        out_shape=jax.ShapeDtypeStruct((1, n), x.dtype),
        in_specs=[pl.BlockSpec((chunk, n), lambda i: (i, 0))],
        out_specs=pl.BlockSpec((1, n), lambda i: (0, 0)),
        scratch_shapes=[
            pltpu.VMEM((N_DEV, 8, n), x.dtype),
            pltpu.SemaphoreType.DMA((N_DEV - 1,)),
            pltpu.SemaphoreType.DMA((N_DEV,)),
        ],
        compiler_params=pltpu.CompilerParams(collective_id=0),
    )(x)


# device time: 8342 ns/iter; 1.0030x vs baseline; 1.0030x over previous
import os

import jax
import jax.numpy as jnp
from jax import lax
from jax.experimental import pallas as pl
from jax.experimental.pallas import tpu as pltpu

N_DEV = 4
K = 8

_MODE = os.environ.get("SCB_MODE", "full")


def kernel(x):
    m_per, n = x.shape
    k = 1 if _MODE == "comm" else K
    chunk = m_per // k

    def body(x_ref, out_ref, comm_ref, send_sems, recv_sems):
        i = pl.program_id(0)
        my_pos = lax.axis_index("i")
        barrier_sem = (
            None if _MODE == "compute" else pltpu.get_barrier_semaphore()
        )

        @pl.when(i == 0)
        def _():
            if _MODE != "compute":
                for d in range(1, N_DEV):
                    peer = lax.rem(my_pos + d, N_DEV)
                    pl.semaphore_signal(
                        barrier_sem, inc=1,
                        device_id=(peer,), device_id_type=pl.DeviceIdType.MESH,
                    )
            comm_ref[0, :, :] = jnp.zeros((8, n), comm_ref.dtype)

        if _MODE != "comm":
            comm_ref[0, :, :] += jnp.sum(
                x_ref[:, :].reshape(chunk // 8, 8, n), axis=0
            )

        @pl.when(i == k - 1)
        def _():
            if _MODE == "compute":
                out_ref[:, :] = jnp.sum(
                    comm_ref[0, :, :], axis=0, keepdims=True
                )
                return
            pl.semaphore_wait(barrier_sem, N_DEV - 1)

            sends = []
            for d in range(1, N_DEV):
                peer = lax.rem(my_pos + d, N_DEV)
                e = N_DEV - d
                rdma = pltpu.make_async_remote_copy(
                    src_ref=comm_ref.at[0],
                    dst_ref=comm_ref.at[e],
                    send_sem=send_sems.at[d - 1],
                    recv_sem=recv_sems.at[e],
                    device_id=(peer,),
                    device_id_type=pl.DeviceIdType.MESH,
                )
                rdma.start()
                sends.append(rdma)

            for e in range(1, N_DEV):
                recv = pltpu.make_async_remote_copy(
                    src_ref=comm_ref.at[0],
                    dst_ref=comm_ref.at[e],
                    send_sem=send_sems.at[0],
                    recv_sem=recv_sems.at[e],
                    device_id=(my_pos,),
                    device_id_type=pl.DeviceIdType.MESH,
                )
                recv.wait_recv()

            total = (
                comm_ref[0, :, :] + comm_ref[1, :, :]
                + comm_ref[2, :, :] + comm_ref[3, :, :]
            )
            out_ref[:, :] = jnp.sum(total, axis=0, keepdims=True)

            for rdma in sends:
                rdma.wait_send()

    if _MODE == "comm":
        in_spec = pl.BlockSpec((8, n), lambda i: (0, 0))
    else:
        in_spec = pl.BlockSpec((chunk, n), lambda i: (i, 0))
    return pl.pallas_call(
        body,
        grid=(k,),
        out_shape=jax.ShapeDtypeStruct((1, n), x.dtype),
        in_specs=[in_spec],
        out_specs=pl.BlockSpec((1, n), lambda i: (0, 0)),
        scratch_shapes=[
            pltpu.VMEM((N_DEV, 8, n), x.dtype),
            pltpu.SemaphoreType.DMA((N_DEV - 1,)),
            pltpu.SemaphoreType.DMA((N_DEV,)),
        ],
        compiler_params=pltpu.CompilerParams(
            collective_id=None if _MODE == "compute" else 0
        ),
    )(x)
